# Initial kernel scaffold; baseline (speedup 1.0000x reference)
import jax
import jax.numpy as jnp
from jax.experimental import pallas as pl


def kernel(x, moe_gate_qkv, moe_w1_qkv, moe_w2_qkv, blk_norm1_w, blk_norm1_b, blk_norm2_w, blk_norm2_b, blk_q_w, blk_k_w, blk_v_w, blk_gate, blk_w1, blk_w2, out_proj_w, norm_out_w, norm_out_b):
    raise NotImplementedError("write your pallas kernel here")


# speedup vs baseline: 1.3782x; 1.3782x over previous
import functools
import math

import jax
import jax.numpy as jnp
from jax.experimental import pallas as pl
from jax.experimental.pallas import tpu as pltpu

NH = 12
LEPS = 1e-5


def _gelu(x):
    return 0.5 * x * (1.0 + jax.lax.erf(x * 0.7071067811865476))


def _layernorm(x, w, b):
    m = jnp.mean(x, axis=-1, keepdims=True)
    v = jnp.mean((x - m) ** 2, axis=-1, keepdims=True)
    return (x - m) * jax.lax.rsqrt(v + LEPS) * w + b


def _top2_combine_col(logits, e):
    E = logits.shape[-1]
    m1 = jnp.max(logits, axis=-1, keepdims=True)
    ii = jax.lax.broadcasted_iota(jnp.int32, logits.shape, 1)
    i1 = jnp.min(jnp.where(logits == m1, ii, E), axis=-1, keepdims=True)
    masked = jnp.where(ii == i1, -1e30, logits)
    m2 = jnp.max(masked, axis=-1, keepdims=True)
    i2 = jnp.min(jnp.where(masked == m2, ii, E), axis=-1, keepdims=True)
    bb = jnp.exp(m2 - m1)
    w1 = 1.0 / (1.0 + bb)
    w2 = bb / (1.0 + bb)
    return jnp.where(i1 == e, w1, 0.0) + jnp.where(i2 == e, w2, 0.0)


def _moe_body(x_ref, gate_ref, w1_ref, w2_ref, *rest, apply_ln, has_res):
    idx = 0
    if apply_ln:
        lnw_ref, lnb_ref = rest[idx], rest[idx + 1]
        idx += 2
    if has_res:
        res_ref = rest[idx]
        idx += 1
    o_ref = rest[idx]

    e = pl.program_id(1)
    x = x_ref[...]
    if apply_ln:
        x = _layernorm(x, lnw_ref[...], lnb_ref[...])
    logits = jax.lax.dot_general(x, gate_ref[...], (((1,), (1,)), ((), ())))
    comb_e = _top2_combine_col(logits, e)
    h = jax.lax.dot_general(x, w1_ref[0], (((1,), (1,)), ((), ())))
    h = _gelu(h)
    eo = jax.lax.dot_general(h, w2_ref[0], (((1,), (1,)), ((), ())))
    contrib = comb_e * eo

    @pl.when(e == 0)
    def _():
        if has_res:
            o_ref[...] = res_ref[...] + contrib
        else:
            o_ref[...] = contrib

    @pl.when(e > 0)
    def _():
        o_ref[...] = o_ref[...] + contrib


def _moe(x, gate, w1, w2, lnw=None, lnb=None, res=None):
    T, D = x.shape
    E, H2, _ = w1.shape
    BT = min(512, T)
    apply_ln = lnw is not None
    has_res = res is not None

    in_specs = [
        pl.BlockSpec((BT, D), lambda t, e: (t, 0)),
        pl.BlockSpec((E, D), lambda t, e: (0, 0)),
        pl.BlockSpec((1, H2, D), lambda t, e: (e, 0, 0)),
        pl.BlockSpec((1, D, H2), lambda t, e: (e, 0, 0)),
    ]
    args = [x, gate, w1, w2]
    if apply_ln:
        in_specs += [pl.BlockSpec((1, D), lambda t, e: (0, 0))] * 2
        args += [lnw.reshape(1, D), lnb.reshape(1, D)]
    if has_res:
        in_specs.append(pl.BlockSpec((BT, D), lambda t, e: (t, 0)))
        args.append(res)

    return pl.pallas_call(
        functools.partial(_moe_body, apply_ln=apply_ln, has_res=has_res),
        grid=(T // BT, E),
        in_specs=in_specs,
        out_specs=pl.BlockSpec((BT, D), lambda t, e: (t, 0)),
        out_shape=jax.ShapeDtypeStruct((T, D), jnp.float32),
    )(*args)


def _attn_body(q_ref, k_ref, v_ref, res_ref, o_ref, *, scale):
    s = jax.lax.dot_general(q_ref[0], k_ref[0],
                            (((1,), (1,)), ((), ()))) * scale
    m = jnp.max(s, axis=-1, keepdims=True)
    p = jnp.exp(s - m)
    l = jnp.sum(p, axis=-1, keepdims=True)
    o = jnp.dot(p, v_ref[0])
    o_ref[0] = res_ref[0] + o / l


def _attn(q, k, v, res):
    _, T, hd = q.shape
    BQ = min(256, T)
    scale = 1.0 / math.sqrt(hd)
    return pl.pallas_call(
        functools.partial(_attn_body, scale=scale),
        grid=(NH, T // BQ),
        in_specs=[
            pl.BlockSpec((1, BQ, hd), lambda h, i: (h, i, 0)),
            pl.BlockSpec((1, T, hd), lambda h, i: (h, 0, 0)),
            pl.BlockSpec((1, T, hd), lambda h, i: (h, 0, 0)),
            pl.BlockSpec((1, BQ, hd), lambda h, i: (h, i, 0)),
        ],
        out_specs=pl.BlockSpec((1, BQ, hd), lambda h, i: (h, i, 0)),
        out_shape=jax.ShapeDtypeStruct((NH, T, hd), jnp.float32),
    )(q, k, v, res)


def _heads(t):
    T, D = t.shape
    return t.reshape(T, NH, D // NH).swapaxes(0, 1)


def _unheads(t):
    NHh, T, hd = t.shape
    return t.swapaxes(0, 1).reshape(T, NHh * hd)


def _lnqkv_body(h_ref, lnw_ref, lnb_ref, wq_ref, wk_ref, wv_ref,
                q_ref, k_ref, v_ref):
    n = _layernorm(h_ref[...], lnw_ref[...], lnb_ref[...])
    cdims = (((1,), (1,)), ((), ()))
    q_ref[...] = jax.lax.dot_general(n, wq_ref[...], cdims)
    k_ref[...] = jax.lax.dot_general(n, wk_ref[...], cdims)
    v_ref[...] = jax.lax.dot_general(n, wv_ref[...], cdims)


def _lnqkv(h, lnw, lnb, wq, wk, wv):
    T, D = h.shape
    BT = min(512, T)
    out = jax.ShapeDtypeStruct((T, D), jnp.float32)
    return pl.pallas_call(
        _lnqkv_body,
        grid=(T // BT,),
        in_specs=[
            pl.BlockSpec((BT, D), lambda t: (t, 0)),
            pl.BlockSpec((1, D), lambda t: (0, 0)),
            pl.BlockSpec((1, D), lambda t: (0, 0)),
            pl.BlockSpec((D, D), lambda t: (0, 0)),
            pl.BlockSpec((D, D), lambda t: (0, 0)),
            pl.BlockSpec((D, D), lambda t: (0, 0)),
        ],
        out_specs=[pl.BlockSpec((BT, D), lambda t: (t, 0))] * 3,
        out_shape=[out, out, out],
    )(h, lnw.reshape(1, D), lnb.reshape(1, D), wq, wk, wv)


def _final_body(h_ref, wo_ref, lnw_ref, lnb_ref, o_ref):
    m = jnp.mean(h_ref[...], axis=0, keepdims=True)
    o = jax.lax.dot_general(m, wo_ref[...], (((1,), (1,)), ((), ())))
    o_ref[...] = _layernorm(o, lnw_ref[...], lnb_ref[...])


def _finalize(h, wo, lnw, lnb):
    OUT = wo.shape[0]
    return pl.pallas_call(
        _final_body,
        out_shape=jax.ShapeDtypeStruct((1, OUT), jnp.float32),
    )(h, wo, lnw.reshape(1, OUT), lnb.reshape(1, OUT))


def kernel(x, moe_gate_qkv, moe_w1_qkv, moe_w2_qkv, blk_norm1_w, blk_norm1_b,
           blk_norm2_w, blk_norm2_b, blk_q_w, blk_k_w, blk_v_w, blk_gate,
           blk_w1, blk_w2, out_proj_w, norm_out_w, norm_out_b):
    x2 = x[0]
    L = blk_q_w.shape[0]

    q = _moe(x2, moe_gate_qkv[0], moe_w1_qkv[0], moe_w2_qkv[0])
    k = _moe(x2, moe_gate_qkv[1], moe_w1_qkv[1], moe_w2_qkv[1])
    v = _moe(x2, moe_gate_qkv[2], moe_w1_qkv[2], moe_w2_qkv[2])
    h = _unheads(_attn(_heads(q), _heads(k), _heads(v), _heads(x2)))
    for l in range(L):
        ql, kl, vl = _lnqkv(h, blk_norm1_w[l], blk_norm1_b[l],
                            blk_q_w[l], blk_k_w[l], blk_v_w[l])
        h = _unheads(_attn(_heads(ql), _heads(kl), _heads(vl), _heads(h)))
        h = _moe(h, blk_gate[l], blk_w1[l], blk_w2[l],
                 lnw=blk_norm2_w[l], lnb=blk_norm2_b[l], res=h)
    return _finalize(h, out_proj_w, norm_out_w, norm_out_b)


# speedup vs baseline: 1.5069x; 1.0934x over previous
import functools
import math

import jax
import jax.numpy as jnp
from jax.experimental import pallas as pl
from jax.experimental.pallas import tpu as pltpu

NH = 12
LEPS = 1e-5


def _bdot(a, b, dims):
    return jax.lax.dot_general(a.astype(jnp.bfloat16), b.astype(jnp.bfloat16),
                               dims, preferred_element_type=jnp.float32)


def _gelu(x):
    return 0.5 * x * (1.0 + jax.lax.erf(x * 0.7071067811865476))


def _layernorm(x, w, b):
    m = jnp.mean(x, axis=-1, keepdims=True)
    v = jnp.mean((x - m) ** 2, axis=-1, keepdims=True)
    return (x - m) * jax.lax.rsqrt(v + LEPS) * w + b


def _top2_combine_col(logits, e):
    E = logits.shape[-1]
    m1 = jnp.max(logits, axis=-1, keepdims=True)
    ii = jax.lax.broadcasted_iota(jnp.int32, logits.shape, 1)
    i1 = jnp.min(jnp.where(logits == m1, ii, E), axis=-1, keepdims=True)
    masked = jnp.where(ii == i1, -1e30, logits)
    m2 = jnp.max(masked, axis=-1, keepdims=True)
    i2 = jnp.min(jnp.where(masked == m2, ii, E), axis=-1, keepdims=True)
    bb = jnp.exp(m2 - m1)
    w1 = 1.0 / (1.0 + bb)
    w2 = bb / (1.0 + bb)
    return jnp.where(i1 == e, w1, 0.0) + jnp.where(i2 == e, w2, 0.0)


def _moe_body(x_ref, gate_ref, w1_ref, w2_ref, *rest, apply_ln, has_res):
    idx = 0
    if apply_ln:
        lnw_ref, lnb_ref = rest[idx], rest[idx + 1]
        idx += 2
    if has_res:
        res_ref = rest[idx]
        idx += 1
    o_ref = rest[idx]

    e = pl.program_id(1)
    x = x_ref[...]
    if apply_ln:
        x = _layernorm(x, lnw_ref[...], lnb_ref[...])
    logits = jax.lax.dot_general(x, gate_ref[...], (((1,), (1,)), ((), ())))
    comb_e = _top2_combine_col(logits, e)
    h = _bdot(x, w1_ref[0], (((1,), (1,)), ((), ())))
    h = _gelu(h)
    eo = _bdot(h, w2_ref[0], (((1,), (1,)), ((), ())))
    contrib = comb_e * eo

    @pl.when(e == 0)
    def _():
        if has_res:
            o_ref[...] = res_ref[...] + contrib
        else:
            o_ref[...] = contrib

    @pl.when(e > 0)
    def _():
        o_ref[...] = o_ref[...] + contrib


def _moe(x, gate, w1, w2, lnw=None, lnb=None, res=None):
    T, D = x.shape
    E, H2, _ = w1.shape
    BT = min(512, T)
    apply_ln = lnw is not None
    has_res = res is not None

    in_specs = [
        pl.BlockSpec((BT, D), lambda t, e: (t, 0)),
        pl.BlockSpec((E, D), lambda t, e: (0, 0)),
        pl.BlockSpec((1, H2, D), lambda t, e: (e, 0, 0)),
        pl.BlockSpec((1, D, H2), lambda t, e: (e, 0, 0)),
    ]
    args = [x, gate, w1, w2]
    if apply_ln:
        in_specs += [pl.BlockSpec((1, D), lambda t, e: (0, 0))] * 2
        args += [lnw.reshape(1, D), lnb.reshape(1, D)]
    if has_res:
        in_specs.append(pl.BlockSpec((BT, D), lambda t, e: (t, 0)))
        args.append(res)

    return pl.pallas_call(
        functools.partial(_moe_body, apply_ln=apply_ln, has_res=has_res),
        grid=(T // BT, E),
        in_specs=in_specs,
        out_specs=pl.BlockSpec((BT, D), lambda t, e: (t, 0)),
        out_shape=jax.ShapeDtypeStruct((T, D), jnp.float32),
    )(*args)


def _attn_body(q_ref, k_ref, v_ref, res_ref, o_ref, *, scale):
    s = _bdot(q_ref[0], k_ref[0], (((1,), (1,)), ((), ()))) * scale
    m = jnp.max(s, axis=-1, keepdims=True)
    p = jnp.exp(s - m)
    l = jnp.sum(p, axis=-1, keepdims=True)
    o = _bdot(p, v_ref[0], (((1,), (0,)), ((), ())))
    o_ref[0] = res_ref[0] + o / l


def _attn(q, k, v, res):
    _, T, hd = q.shape
    BQ = min(256, T)
    scale = 1.0 / math.sqrt(hd)
    return pl.pallas_call(
        functools.partial(_attn_body, scale=scale),
        grid=(NH, T // BQ),
        in_specs=[
            pl.BlockSpec((1, BQ, hd), lambda h, i: (h, i, 0)),
            pl.BlockSpec((1, T, hd), lambda h, i: (h, 0, 0)),
            pl.BlockSpec((1, T, hd), lambda h, i: (h, 0, 0)),
            pl.BlockSpec((1, BQ, hd), lambda h, i: (h, i, 0)),
        ],
        out_specs=pl.BlockSpec((1, BQ, hd), lambda h, i: (h, i, 0)),
        out_shape=jax.ShapeDtypeStruct((NH, T, hd), jnp.float32),
    )(q, k, v, res)


def _heads(t):
    T, D = t.shape
    return t.reshape(T, NH, D // NH).swapaxes(0, 1)


def _unheads(t):
    NHh, T, hd = t.shape
    return t.swapaxes(0, 1).reshape(T, NHh * hd)


def _lnqkv_body(h_ref, lnw_ref, lnb_ref, wq_ref, wk_ref, wv_ref,
                q_ref, k_ref, v_ref):
    n = _layernorm(h_ref[...], lnw_ref[...], lnb_ref[...])
    cdims = (((1,), (1,)), ((), ()))
    n16 = n.astype(jnp.bfloat16)
    q_ref[...] = _bdot(n16, wq_ref[...], cdims)
    k_ref[...] = _bdot(n16, wk_ref[...], cdims)
    v_ref[...] = _bdot(n16, wv_ref[...], cdims)


def _lnqkv(h, lnw, lnb, wq, wk, wv):
    T, D = h.shape
    BT = min(512, T)
    out = jax.ShapeDtypeStruct((T, D), jnp.float32)
    return pl.pallas_call(
        _lnqkv_body,
        grid=(T // BT,),
        in_specs=[
            pl.BlockSpec((BT, D), lambda t: (t, 0)),
            pl.BlockSpec((1, D), lambda t: (0, 0)),
            pl.BlockSpec((1, D), lambda t: (0, 0)),
            pl.BlockSpec((D, D), lambda t: (0, 0)),
            pl.BlockSpec((D, D), lambda t: (0, 0)),
            pl.BlockSpec((D, D), lambda t: (0, 0)),
        ],
        out_specs=[pl.BlockSpec((BT, D), lambda t: (t, 0))] * 3,
        out_shape=[out, out, out],
    )(h, lnw.reshape(1, D), lnb.reshape(1, D), wq, wk, wv)


def _final_body(h_ref, wo_ref, lnw_ref, lnb_ref, o_ref):
    m = jnp.mean(h_ref[...], axis=0, keepdims=True)
    o = jax.lax.dot_general(m, wo_ref[...], (((1,), (1,)), ((), ())))
    o_ref[...] = _layernorm(o, lnw_ref[...], lnb_ref[...])


def _finalize(h, wo, lnw, lnb):
    OUT = wo.shape[0]
    return pl.pallas_call(
        _final_body,
        out_shape=jax.ShapeDtypeStruct((1, OUT), jnp.float32),
    )(h, wo, lnw.reshape(1, OUT), lnb.reshape(1, OUT))


def kernel(x, moe_gate_qkv, moe_w1_qkv, moe_w2_qkv, blk_norm1_w, blk_norm1_b,
           blk_norm2_w, blk_norm2_b, blk_q_w, blk_k_w, blk_v_w, blk_gate,
           blk_w1, blk_w2, out_proj_w, norm_out_w, norm_out_b):
    x2 = x[0]
    L = blk_q_w.shape[0]

    q = _moe(x2, moe_gate_qkv[0], moe_w1_qkv[0], moe_w2_qkv[0])
    k = _moe(x2, moe_gate_qkv[1], moe_w1_qkv[1], moe_w2_qkv[1])
    v = _moe(x2, moe_gate_qkv[2], moe_w1_qkv[2], moe_w2_qkv[2])
    h = _unheads(_attn(_heads(q), _heads(k), _heads(v), _heads(x2)))
    for l in range(L):
        ql, kl, vl = _lnqkv(h, blk_norm1_w[l], blk_norm1_b[l],
                            blk_q_w[l], blk_k_w[l], blk_v_w[l])
        h = _unheads(_attn(_heads(ql), _heads(kl), _heads(vl), _heads(h)))
        h = _moe(h, blk_gate[l], blk_w1[l], blk_w2[l],
                 lnw=blk_norm2_w[l], lnb=blk_norm2_b[l], res=h)
    return _finalize(h, out_proj_w, norm_out_w, norm_out_b)


# speedup vs baseline: 1.5589x; 1.0345x over previous
import functools
import math

import jax
import jax.numpy as jnp
from jax.experimental import pallas as pl
from jax.experimental.pallas import tpu as pltpu

NH = 12
LEPS = 1e-5


def _bdot(a, b, dims):
    return jax.lax.dot_general(a.astype(jnp.bfloat16), b.astype(jnp.bfloat16),
                               dims, preferred_element_type=jnp.float32)


def _gelu(x):
    return 0.5 * x * (1.0 + jax.lax.erf(x * 0.7071067811865476))


def _layernorm(x, w, b):
    m = jnp.mean(x, axis=-1, keepdims=True)
    v = jnp.mean((x - m) ** 2, axis=-1, keepdims=True)
    return (x - m) * jax.lax.rsqrt(v + LEPS) * w + b


def _top2_combine_col(logits, e):
    E = logits.shape[-1]
    m1 = jnp.max(logits, axis=-1, keepdims=True)
    ii = jax.lax.broadcasted_iota(jnp.int32, logits.shape, 1)
    i1 = jnp.min(jnp.where(logits == m1, ii, E), axis=-1, keepdims=True)
    masked = jnp.where(ii == i1, -1e30, logits)
    m2 = jnp.max(masked, axis=-1, keepdims=True)
    i2 = jnp.min(jnp.where(masked == m2, ii, E), axis=-1, keepdims=True)
    bb = jnp.exp(m2 - m1)
    w1 = 1.0 / (1.0 + bb)
    w2 = bb / (1.0 + bb)
    return jnp.where(i1 == e, w1, 0.0) + jnp.where(i2 == e, w2, 0.0)


def _moe_body(x_ref, gate_ref, w1_ref, w2_ref, *rest, apply_ln, has_res):
    idx = 0
    if apply_ln:
        lnw_ref, lnb_ref = rest[idx], rest[idx + 1]
        idx += 2
    if has_res:
        res_ref = rest[idx]
        idx += 1
    o_ref = rest[idx]

    e = pl.program_id(1)
    x = x_ref[...]
    if apply_ln:
        x = _layernorm(x, lnw_ref[...], lnb_ref[...])
    logits = jax.lax.dot_general(x, gate_ref[...], (((1,), (1,)), ((), ())))
    comb_e = _top2_combine_col(logits, e)
    h = _bdot(x, w1_ref[0], (((1,), (1,)), ((), ())))
    h = _gelu(h)
    eo = _bdot(h, w2_ref[0], (((1,), (1,)), ((), ())))
    contrib = comb_e * eo

    @pl.when(e == 0)
    def _():
        if has_res:
            o_ref[...] = res_ref[...] + contrib
        else:
            o_ref[...] = contrib

    @pl.when(e > 0)
    def _():
        o_ref[...] = o_ref[...] + contrib


def _moe(x, gate, w1, w2, lnw=None, lnb=None, res=None):
    T, D = x.shape
    E, H2, _ = w1.shape
    BT = min(2048, T)
    apply_ln = lnw is not None
    has_res = res is not None

    in_specs = [
        pl.BlockSpec((BT, D), lambda t, e: (t, 0)),
        pl.BlockSpec((E, D), lambda t, e: (0, 0)),
        pl.BlockSpec((1, H2, D), lambda t, e: (e, 0, 0)),
        pl.BlockSpec((1, D, H2), lambda t, e: (e, 0, 0)),
    ]
    args = [x, gate, w1, w2]
    if apply_ln:
        in_specs += [pl.BlockSpec((1, D), lambda t, e: (0, 0))] * 2
        args += [lnw.reshape(1, D), lnb.reshape(1, D)]
    if has_res:
        in_specs.append(pl.BlockSpec((BT, D), lambda t, e: (t, 0)))
        args.append(res)

    return pl.pallas_call(
        functools.partial(_moe_body, apply_ln=apply_ln, has_res=has_res),
        grid=(T // BT, E),
        in_specs=in_specs,
        out_specs=pl.BlockSpec((BT, D), lambda t, e: (t, 0)),
        out_shape=jax.ShapeDtypeStruct((T, D), jnp.float32),
    )(*args)


def _attn_body(q_ref, k_ref, v_ref, res_ref, o_ref, *, scale):
    s = _bdot(q_ref[0], k_ref[0], (((1,), (1,)), ((), ()))) * scale
    m = jnp.max(s, axis=-1, keepdims=True)
    p = jnp.exp(s - m)
    l = jnp.sum(p, axis=-1, keepdims=True)
    o = _bdot(p, v_ref[0], (((1,), (0,)), ((), ())))
    o_ref[0] = res_ref[0] + o / l


def _attn(q, k, v, res):
    _, T, hd = q.shape
    BQ = min(256, T)
    scale = 1.0 / math.sqrt(hd)
    return pl.pallas_call(
        functools.partial(_attn_body, scale=scale),
        grid=(NH, T // BQ),
        in_specs=[
            pl.BlockSpec((1, BQ, hd), lambda h, i: (h, i, 0)),
            pl.BlockSpec((1, T, hd), lambda h, i: (h, 0, 0)),
            pl.BlockSpec((1, T, hd), lambda h, i: (h, 0, 0)),
            pl.BlockSpec((1, BQ, hd), lambda h, i: (h, i, 0)),
        ],
        out_specs=pl.BlockSpec((1, BQ, hd), lambda h, i: (h, i, 0)),
        out_shape=jax.ShapeDtypeStruct((NH, T, hd), jnp.float32),
    )(q, k, v, res)


def _heads(t):
    T, D = t.shape
    return t.reshape(T, NH, D // NH).swapaxes(0, 1)


def _unheads(t):
    NHh, T, hd = t.shape
    return t.swapaxes(0, 1).reshape(T, NHh * hd)


def _lnqkv_body(h_ref, lnw_ref, lnb_ref, wq_ref, wk_ref, wv_ref,
                q_ref, k_ref, v_ref):
    n = _layernorm(h_ref[...], lnw_ref[...], lnb_ref[...])
    cdims = (((1,), (1,)), ((), ()))
    n16 = n.astype(jnp.bfloat16)
    q_ref[...] = _bdot(n16, wq_ref[...], cdims)
    k_ref[...] = _bdot(n16, wk_ref[...], cdims)
    v_ref[...] = _bdot(n16, wv_ref[...], cdims)


def _lnqkv(h, lnw, lnb, wq, wk, wv):
    T, D = h.shape
    BT = min(512, T)
    out = jax.ShapeDtypeStruct((T, D), jnp.float32)
    return pl.pallas_call(
        _lnqkv_body,
        grid=(T // BT,),
        in_specs=[
            pl.BlockSpec((BT, D), lambda t: (t, 0)),
            pl.BlockSpec((1, D), lambda t: (0, 0)),
            pl.BlockSpec((1, D), lambda t: (0, 0)),
            pl.BlockSpec((D, D), lambda t: (0, 0)),
            pl.BlockSpec((D, D), lambda t: (0, 0)),
            pl.BlockSpec((D, D), lambda t: (0, 0)),
        ],
        out_specs=[pl.BlockSpec((BT, D), lambda t: (t, 0))] * 3,
        out_shape=[out, out, out],
    )(h, lnw.reshape(1, D), lnb.reshape(1, D), wq, wk, wv)


def _final_body(h_ref, wo_ref, lnw_ref, lnb_ref, o_ref):
    m = jnp.mean(h_ref[...], axis=0, keepdims=True)
    o = jax.lax.dot_general(m, wo_ref[...], (((1,), (1,)), ((), ())))
    o_ref[...] = _layernorm(o, lnw_ref[...], lnb_ref[...])


def _finalize(h, wo, lnw, lnb):
    OUT = wo.shape[0]
    return pl.pallas_call(
        _final_body,
        out_shape=jax.ShapeDtypeStruct((1, OUT), jnp.float32),
    )(h, wo, lnw.reshape(1, OUT), lnb.reshape(1, OUT))


def kernel(x, moe_gate_qkv, moe_w1_qkv, moe_w2_qkv, blk_norm1_w, blk_norm1_b,
           blk_norm2_w, blk_norm2_b, blk_q_w, blk_k_w, blk_v_w, blk_gate,
           blk_w1, blk_w2, out_proj_w, norm_out_w, norm_out_b):
    x2 = x[0]
    L = blk_q_w.shape[0]

    q = _moe(x2, moe_gate_qkv[0], moe_w1_qkv[0], moe_w2_qkv[0])
    k = _moe(x2, moe_gate_qkv[1], moe_w1_qkv[1], moe_w2_qkv[1])
    v = _moe(x2, moe_gate_qkv[2], moe_w1_qkv[2], moe_w2_qkv[2])
    h = _unheads(_attn(_heads(q), _heads(k), _heads(v), _heads(x2)))
    for l in range(L):
        ql, kl, vl = _lnqkv(h, blk_norm1_w[l], blk_norm1_b[l],
                            blk_q_w[l], blk_k_w[l], blk_v_w[l])
        h = _unheads(_attn(_heads(ql), _heads(kl), _heads(vl), _heads(h)))
        h = _moe(h, blk_gate[l], blk_w1[l], blk_w2[l],
                 lnw=blk_norm2_w[l], lnb=blk_norm2_b[l], res=h)
    return _finalize(h, out_proj_w, norm_out_w, norm_out_b)


# speedup vs baseline: 1.5750x; 1.0103x over previous
import functools
import math

import jax
import jax.numpy as jnp
from jax.experimental import pallas as pl
from jax.experimental.pallas import tpu as pltpu

NH = 12
LEPS = 1e-5


def _bdot(a, b, dims):
    return jax.lax.dot_general(a.astype(jnp.bfloat16), b.astype(jnp.bfloat16),
                               dims, preferred_element_type=jnp.float32)


def _gelu(x):
    return 0.5 * x * (1.0 + jax.lax.erf(x * 0.7071067811865476))


def _layernorm(x, w, b):
    m = jnp.mean(x, axis=-1, keepdims=True)
    v = jnp.mean((x - m) ** 2, axis=-1, keepdims=True)
    return (x - m) * jax.lax.rsqrt(v + LEPS) * w + b


def _top2_combine_col(logits, e):
    E = logits.shape[-1]
    m1 = jnp.max(logits, axis=-1, keepdims=True)
    ii = jax.lax.broadcasted_iota(jnp.int32, logits.shape, 1)
    i1 = jnp.min(jnp.where(logits == m1, ii, E), axis=-1, keepdims=True)
    masked = jnp.where(ii == i1, -1e30, logits)
    m2 = jnp.max(masked, axis=-1, keepdims=True)
    i2 = jnp.min(jnp.where(masked == m2, ii, E), axis=-1, keepdims=True)
    bb = jnp.exp(m2 - m1)
    w1 = 1.0 / (1.0 + bb)
    w2 = bb / (1.0 + bb)
    return jnp.where(i1 == e, w1, 0.0) + jnp.where(i2 == e, w2, 0.0)


def _moe_body(x_ref, gate_ref, w1_ref, w2_ref, *rest, apply_ln, has_res):
    idx = 0
    if apply_ln:
        lnw_ref, lnb_ref = rest[idx], rest[idx + 1]
        idx += 2
    if has_res:
        res_ref = rest[idx]
        idx += 1
    o_ref = rest[idx]

    e = pl.program_id(1)
    x = x_ref[...]
    if apply_ln:
        x = _layernorm(x, lnw_ref[...], lnb_ref[...])
    logits = jax.lax.dot_general(x, gate_ref[...], (((1,), (1,)), ((), ())))
    comb_e = _top2_combine_col(logits, e)
    h = _bdot(x, w1_ref[0], (((1,), (1,)), ((), ())))
    h = _gelu(h)
    eo = _bdot(h, w2_ref[0], (((1,), (1,)), ((), ())))
    contrib = comb_e * eo

    @pl.when(e == 0)
    def _():
        if has_res:
            o_ref[...] = res_ref[...] + contrib
        else:
            o_ref[...] = contrib

    @pl.when(e > 0)
    def _():
        o_ref[...] = o_ref[...] + contrib


def _moe(x, gate, w1, w2, lnw=None, lnb=None, res=None):
    T, D = x.shape
    E, H2, _ = w1.shape
    BT = min(2048, T)
    apply_ln = lnw is not None
    has_res = res is not None

    in_specs = [
        pl.BlockSpec((BT, D), lambda t, e: (t, 0)),
        pl.BlockSpec((E, D), lambda t, e: (0, 0)),
        pl.BlockSpec((1, H2, D), lambda t, e: (e, 0, 0)),
        pl.BlockSpec((1, D, H2), lambda t, e: (e, 0, 0)),
    ]
    args = [x, gate, w1, w2]
    if apply_ln:
        in_specs += [pl.BlockSpec((1, D), lambda t, e: (0, 0))] * 2
        args += [lnw.reshape(1, D), lnb.reshape(1, D)]
    if has_res:
        in_specs.append(pl.BlockSpec((BT, D), lambda t, e: (t, 0)))
        args.append(res)

    return pl.pallas_call(
        functools.partial(_moe_body, apply_ln=apply_ln, has_res=has_res),
        grid=(T // BT, E),
        in_specs=in_specs,
        out_specs=pl.BlockSpec((BT, D), lambda t, e: (t, 0)),
        out_shape=jax.ShapeDtypeStruct((T, D), jnp.float32),
    )(*args)


def _attn_body(q_ref, k_ref, v_ref, res_ref, o_ref, *, scale):
    s = _bdot(q_ref[0], k_ref[0], (((1,), (1,)), ((), ()))) * scale
    m = jnp.max(s, axis=-1, keepdims=True)
    p = jnp.exp(s - m)
    l = jnp.sum(p, axis=-1, keepdims=True)
    o = _bdot(p, v_ref[0], (((1,), (0,)), ((), ())))
    o_ref[0] = res_ref[0] + o / l


def _attn(q, k, v, res):
    _, T, hd = q.shape
    BQ = min(256, T)
    scale = 1.0 / math.sqrt(hd)
    return pl.pallas_call(
        functools.partial(_attn_body, scale=scale),
        grid=(NH, T // BQ),
        in_specs=[
            pl.BlockSpec((1, BQ, hd), lambda h, i: (h, i, 0)),
            pl.BlockSpec((1, T, hd), lambda h, i: (h, 0, 0)),
            pl.BlockSpec((1, T, hd), lambda h, i: (h, 0, 0)),
            pl.BlockSpec((1, BQ, hd), lambda h, i: (h, i, 0)),
        ],
        out_specs=pl.BlockSpec((1, BQ, hd), lambda h, i: (h, i, 0)),
        out_shape=jax.ShapeDtypeStruct((NH, T, hd), jnp.float32),
    )(q, k, v, res)


def _heads(t, dtype=jnp.bfloat16):
    T, D = t.shape
    return t.astype(dtype).reshape(T, NH, D // NH).swapaxes(0, 1)


def _unheads(t):
    NHh, T, hd = t.shape
    return t.swapaxes(0, 1).reshape(T, NHh * hd)


def _lnqkv_body(h_ref, lnw_ref, lnb_ref, wq_ref, wk_ref, wv_ref,
                q_ref, k_ref, v_ref):
    n = _layernorm(h_ref[...], lnw_ref[...], lnb_ref[...])
    cdims = (((1,), (1,)), ((), ()))
    n16 = n.astype(jnp.bfloat16)
    q_ref[...] = _bdot(n16, wq_ref[...], cdims)
    k_ref[...] = _bdot(n16, wk_ref[...], cdims)
    v_ref[...] = _bdot(n16, wv_ref[...], cdims)


def _lnqkv(h, lnw, lnb, wq, wk, wv):
    T, D = h.shape
    BT = min(512, T)
    out = jax.ShapeDtypeStruct((T, D), jnp.float32)
    return pl.pallas_call(
        _lnqkv_body,
        grid=(T // BT,),
        in_specs=[
            pl.BlockSpec((BT, D), lambda t: (t, 0)),
            pl.BlockSpec((1, D), lambda t: (0, 0)),
            pl.BlockSpec((1, D), lambda t: (0, 0)),
            pl.BlockSpec((D, D), lambda t: (0, 0)),
            pl.BlockSpec((D, D), lambda t: (0, 0)),
            pl.BlockSpec((D, D), lambda t: (0, 0)),
        ],
        out_specs=[pl.BlockSpec((BT, D), lambda t: (t, 0))] * 3,
        out_shape=[out, out, out],
    )(h, lnw.reshape(1, D), lnb.reshape(1, D), wq, wk, wv)


def _final_body(h_ref, wo_ref, lnw_ref, lnb_ref, o_ref):
    m = jnp.mean(h_ref[...], axis=0, keepdims=True)
    o = jax.lax.dot_general(m, wo_ref[...], (((1,), (1,)), ((), ())))
    o_ref[...] = _layernorm(o, lnw_ref[...], lnb_ref[...])


def _finalize(h, wo, lnw, lnb):
    OUT = wo.shape[0]
    return pl.pallas_call(
        _final_body,
        out_shape=jax.ShapeDtypeStruct((1, OUT), jnp.float32),
    )(h, wo, lnw.reshape(1, OUT), lnb.reshape(1, OUT))


def kernel(x, moe_gate_qkv, moe_w1_qkv, moe_w2_qkv, blk_norm1_w, blk_norm1_b,
           blk_norm2_w, blk_norm2_b, blk_q_w, blk_k_w, blk_v_w, blk_gate,
           blk_w1, blk_w2, out_proj_w, norm_out_w, norm_out_b):
    x2 = x[0]
    L = blk_q_w.shape[0]

    q = _moe(x2, moe_gate_qkv[0], moe_w1_qkv[0], moe_w2_qkv[0])
    k = _moe(x2, moe_gate_qkv[1], moe_w1_qkv[1], moe_w2_qkv[1])
    v = _moe(x2, moe_gate_qkv[2], moe_w1_qkv[2], moe_w2_qkv[2])
    h = _unheads(_attn(_heads(q), _heads(k), _heads(v),
                       _heads(x2, jnp.float32)))
    for l in range(L):
        ql, kl, vl = _lnqkv(h, blk_norm1_w[l], blk_norm1_b[l],
                            blk_q_w[l], blk_k_w[l], blk_v_w[l])
        h = _unheads(_attn(_heads(ql), _heads(kl), _heads(vl),
                           _heads(h, jnp.float32)))
        h = _moe(h, blk_gate[l], blk_w1[l], blk_w2[l],
                 lnw=blk_norm2_w[l], lnb=blk_norm2_b[l], res=h)
    return _finalize(h, out_proj_w, norm_out_w, norm_out_b)


# speedup vs baseline: 1.6525x; 1.0492x over previous
import functools
import math

import jax
import jax.numpy as jnp
from jax.experimental import pallas as pl
from jax.experimental.pallas import tpu as pltpu

NH = 12
LEPS = 1e-5


def _bdot(a, b, dims):
    return jax.lax.dot_general(a.astype(jnp.bfloat16), b.astype(jnp.bfloat16),
                               dims, preferred_element_type=jnp.float32)


def _gelu(x):
    return 0.5 * x * (1.0 + jax.lax.erf(x * 0.7071067811865476))


def _layernorm(x, w, b):
    m = jnp.mean(x, axis=-1, keepdims=True)
    v = jnp.mean((x - m) ** 2, axis=-1, keepdims=True)
    return (x - m) * jax.lax.rsqrt(v + LEPS) * w + b


def _top2_combine_col(logits, e):
    E = logits.shape[-1]
    m1 = jnp.max(logits, axis=-1, keepdims=True)
    ii = jax.lax.broadcasted_iota(jnp.int32, logits.shape, 1)
    i1 = jnp.min(jnp.where(logits == m1, ii, E), axis=-1, keepdims=True)
    masked = jnp.where(ii == i1, -1e30, logits)
    m2 = jnp.max(masked, axis=-1, keepdims=True)
    i2 = jnp.min(jnp.where(masked == m2, ii, E), axis=-1, keepdims=True)
    bb = jnp.exp(m2 - m1)
    w1 = 1.0 / (1.0 + bb)
    w2 = bb / (1.0 + bb)
    return jnp.where(i1 == e, w1, 0.0) + jnp.where(i2 == e, w2, 0.0)


def _moe_body(x_ref, gate_ref, w1_ref, w2_ref, *rest, apply_ln, has_res):
    idx = 0
    if apply_ln:
        lnw_ref, lnb_ref = rest[idx], rest[idx + 1]
        idx += 2
    if has_res:
        res_ref = rest[idx]
        idx += 1
    o_ref = rest[idx]

    e = pl.program_id(1)
    x = x_ref[...]
    if apply_ln:
        x = _layernorm(x, lnw_ref[...], lnb_ref[...])
    logits = jax.lax.dot_general(x, gate_ref[...], (((1,), (1,)), ((), ())))
    comb_e = _top2_combine_col(logits, e)
    h = _bdot(x, w1_ref[0], (((1,), (1,)), ((), ())))
    h = _gelu(h)
    eo = _bdot(h, w2_ref[0], (((1,), (1,)), ((), ())))
    contrib = comb_e * eo

    @pl.when(e == 0)
    def _():
        if has_res:
            o_ref[...] = res_ref[...] + contrib
        else:
            o_ref[...] = contrib

    @pl.when(e > 0)
    def _():
        o_ref[...] = o_ref[...] + contrib


def _moe(x, gate, w1, w2, lnw=None, lnb=None, res=None):
    T, D = x.shape
    E, H2, _ = w1.shape
    BT = min(2048, T)
    apply_ln = lnw is not None
    has_res = res is not None

    in_specs = [
        pl.BlockSpec((BT, D), lambda t, e: (t, 0)),
        pl.BlockSpec((E, D), lambda t, e: (0, 0)),
        pl.BlockSpec((1, H2, D), lambda t, e: (e, 0, 0)),
        pl.BlockSpec((1, D, H2), lambda t, e: (e, 0, 0)),
    ]
    args = [x, gate, w1, w2]
    if apply_ln:
        in_specs += [pl.BlockSpec((1, D), lambda t, e: (0, 0))] * 2
        args += [lnw.reshape(1, D), lnb.reshape(1, D)]
    if has_res:
        in_specs.append(pl.BlockSpec((BT, D), lambda t, e: (t, 0)))
        args.append(res)

    return pl.pallas_call(
        functools.partial(_moe_body, apply_ln=apply_ln, has_res=has_res),
        grid=(T // BT, E),
        in_specs=in_specs,
        out_specs=pl.BlockSpec((BT, D), lambda t, e: (t, 0)),
        out_shape=jax.ShapeDtypeStruct((T, D), jnp.float32),
    )(*args)


def _attn_body(q_ref, k_ref, v_ref, res_ref, o_ref, *, scale):
    s = _bdot(q_ref[0], k_ref[0], (((1,), (1,)), ((), ()))) * scale
    m = jnp.max(s, axis=-1, keepdims=True)
    p = jnp.exp(s - m)
    l = jnp.sum(p, axis=-1, keepdims=True)
    o = _bdot(p, v_ref[0], (((1,), (0,)), ((), ())))
    o_ref[0] = res_ref[0] + o / l


def _attn(q, k, v, res):
    _, T, hd = q.shape
    BQ = min(256, T)
    scale = 1.0 / math.sqrt(hd)
    return pl.pallas_call(
        functools.partial(_attn_body, scale=scale),
        grid=(NH, T // BQ),
        in_specs=[
            pl.BlockSpec((1, BQ, hd), lambda h, i: (h, i, 0)),
            pl.BlockSpec((1, T, hd), lambda h, i: (h, 0, 0)),
            pl.BlockSpec((1, T, hd), lambda h, i: (h, 0, 0)),
            pl.BlockSpec((1, BQ, hd), lambda h, i: (h, i, 0)),
        ],
        out_specs=pl.BlockSpec((1, BQ, hd), lambda h, i: (h, i, 0)),
        out_shape=jax.ShapeDtypeStruct((NH, T, hd), jnp.float32),
    )(q, k, v, res)


def _heads(t, dtype=jnp.bfloat16):
    T, D = t.shape
    return t.astype(dtype).reshape(T, NH, D // NH).swapaxes(0, 1)


def _unheads(t):
    NHh, T, hd = t.shape
    return t.swapaxes(0, 1).reshape(T, NHh * hd)


def _lnqkv_body(h_ref, lnw_ref, lnb_ref, wq_ref, wk_ref, wv_ref,
                q_ref, k_ref, v_ref):
    n = _layernorm(h_ref[...], lnw_ref[...], lnb_ref[...])
    cdims = (((1,), (1,)), ((), ()))
    n16 = n.astype(jnp.bfloat16)
    q_ref[...] = _bdot(n16, wq_ref[...], cdims).astype(jnp.bfloat16)
    k_ref[...] = _bdot(n16, wk_ref[...], cdims).astype(jnp.bfloat16)
    v_ref[...] = _bdot(n16, wv_ref[...], cdims).astype(jnp.bfloat16)


def _lnqkv(h, lnw, lnb, wq, wk, wv):
    T, D = h.shape
    BT = min(512, T)
    out = jax.ShapeDtypeStruct((T, D), jnp.bfloat16)
    return pl.pallas_call(
        _lnqkv_body,
        grid=(T // BT,),
        in_specs=[
            pl.BlockSpec((BT, D), lambda t: (t, 0)),
            pl.BlockSpec((1, D), lambda t: (0, 0)),
            pl.BlockSpec((1, D), lambda t: (0, 0)),
            pl.BlockSpec((D, D), lambda t: (0, 0)),
            pl.BlockSpec((D, D), lambda t: (0, 0)),
            pl.BlockSpec((D, D), lambda t: (0, 0)),
        ],
        out_specs=[pl.BlockSpec((BT, D), lambda t: (t, 0))] * 3,
        out_shape=[out, out, out],
    )(h, lnw.reshape(1, D), lnb.reshape(1, D), wq, wk, wv)


def _final_body(h_ref, wo_ref, lnw_ref, lnb_ref, o_ref):
    m = jnp.mean(h_ref[...], axis=0, keepdims=True)
    o = jax.lax.dot_general(m, wo_ref[...], (((1,), (1,)), ((), ())))
    o_ref[...] = _layernorm(o, lnw_ref[...], lnb_ref[...])


def _finalize(h, wo, lnw, lnb):
    OUT = wo.shape[0]
    return pl.pallas_call(
        _final_body,
        out_shape=jax.ShapeDtypeStruct((1, OUT), jnp.float32),
    )(h, wo, lnw.reshape(1, OUT), lnb.reshape(1, OUT))


def kernel(x, moe_gate_qkv, moe_w1_qkv, moe_w2_qkv, blk_norm1_w, blk_norm1_b,
           blk_norm2_w, blk_norm2_b, blk_q_w, blk_k_w, blk_v_w, blk_gate,
           blk_w1, blk_w2, out_proj_w, norm_out_w, norm_out_b):
    x2 = x[0]
    L = blk_q_w.shape[0]

    q = _moe(x2, moe_gate_qkv[0], moe_w1_qkv[0], moe_w2_qkv[0])
    k = _moe(x2, moe_gate_qkv[1], moe_w1_qkv[1], moe_w2_qkv[1])
    v = _moe(x2, moe_gate_qkv[2], moe_w1_qkv[2], moe_w2_qkv[2])
    h = _unheads(_attn(_heads(q), _heads(k), _heads(v),
                       _heads(x2, jnp.float32)))
    for l in range(L):
        ql, kl, vl = _lnqkv(h, blk_norm1_w[l], blk_norm1_b[l],
                            blk_q_w[l], blk_k_w[l], blk_v_w[l])
        h = _unheads(_attn(_heads(ql), _heads(kl), _heads(vl),
                           _heads(h, jnp.float32)))
        h = _moe(h, blk_gate[l], blk_w1[l], blk_w2[l],
                 lnw=blk_norm2_w[l], lnb=blk_norm2_b[l], res=h)
    return _finalize(h, out_proj_w, norm_out_w, norm_out_b)
